# baseline (speedup 1.0000x reference)
import functools

import jax
import jax.numpy as jnp
from jax import lax
from jax.experimental import pallas as pl
from jax.experimental.pallas import tpu as pltpu
from jax.experimental.pallas import tpu_sc as plsc

EMBED_DIM = 32
BATCH = 16384
HIST = 50

NUM_CORES = 2
NUM_SUBCORES = 16
NW = NUM_CORES * NUM_SUBCORES

NCHUNK = 4
CHUNK = BATCH // NCHUNK
RPW = CHUNK // NW
RPG = 8
NG = RPW // RPG


@functools.cache
def _make_kernel():
    mesh = plsc.VectorSubcoreMesh(
        core_axis_name="c", subcore_axis_name="s",
        num_cores=NUM_CORES, num_subcores=NUM_SUBCORES)

    @functools.partial(
        pl.kernel,
        out_type=jax.ShapeDtypeStruct((CHUNK, HIST, EMBED_DIM), jnp.float32),
        mesh=mesh,
        compiler_params=pltpu.CompilerParams(use_tc_tiling_on_sc=False),
        scratch_types=[
            pltpu.VMEM((RPW, HIST), jnp.int32),
            pltpu.VMEM((RPG, HIST, EMBED_DIM), jnp.float32),
            pltpu.VMEM((RPG, HIST, EMBED_DIM), jnp.float32),
            pltpu.SemaphoreType.DMA,
            pltpu.SemaphoreType.DMA,
            pltpu.SemaphoreType.DMA,
            pltpu.SemaphoreType.DMA,
        ],
    )
    def gather_kernel(idx_hbm, table_hbm, out_hbm,
                      idx_v, rows0, rows1, gsem0, gsem1, osem0, osem1):
        wid = lax.axis_index("s") * NUM_CORES + lax.axis_index("c")
        base = wid * RPW
        pltpu.sync_copy(idx_hbm.at[pl.ds(base, RPW)], idx_v)

        def start_gathers(g, rows, gsem):
            for r in range(RPG):
                pltpu.async_copy(
                    table_hbm.at[idx_v.at[g * RPG + r]],
                    rows.at[r],
                    gsem)

        def wait_gathers(rows, gsem):
            pltpu.make_async_copy(out_hbm.at[pl.ds(0, RPG)], rows, gsem).wait()

        def start_out(g, rows, osem):
            pltpu.async_copy(
                rows, out_hbm.at[pl.ds(base + g * RPG, RPG)], osem)

        def wait_out(rows, osem):
            pltpu.make_async_copy(rows, out_hbm.at[pl.ds(0, RPG)], osem).wait()

        start_gathers(0, rows0, gsem0)
        start_gathers(1, rows1, gsem1)
        wait_gathers(rows0, gsem0)
        start_out(0, rows0, osem0)

        def pair_body(p, carry):
            g = 2 * p + 1
            wait_out(rows0, osem0)
            start_gathers(g + 1, rows0, gsem0)
            wait_gathers(rows1, gsem1)
            start_out(g, rows1, osem1)

            wait_out(rows1, osem1)
            start_gathers(g + 2, rows1, gsem1)
            wait_gathers(rows0, gsem0)
            start_out(g + 1, rows0, osem0)
            return carry

        lax.fori_loop(0, (NG - 2) // 2, pair_body, 0)

        wait_out(rows0, osem0)
        wait_gathers(rows1, gsem1)
        start_out(NG - 1, rows1, osem1)
        wait_out(rows1, osem1)

    return gather_kernel


def kernel(indices, table, dummy):
    k = _make_kernel()
    zero = dummy.sum() * 0.0
    outs = [k(indices[c * CHUNK:(c + 1) * CHUNK], table) + zero
            for c in range(NCHUNK)]
    return jnp.concatenate(outs, axis=0)


# speedup vs baseline: 1.9008x; 1.9008x over previous
import functools

import jax
import jax.numpy as jnp
from jax import lax
from jax.experimental import pallas as pl
from jax.experimental.pallas import tpu as pltpu
from jax.experimental.pallas import tpu_sc as plsc

EMBED_DIM = 32
BATCH = 16384
HIST = 50

NUM_CORES = 2
NUM_SUBCORES = 16
NW = NUM_CORES * NUM_SUBCORES

NCHUNK = 4
CHUNK = BATCH // NCHUNK
RPW = CHUNK // NW
RPG = 8
NG = RPW // RPG


@functools.cache
def _make_kernel(chunk):
    mesh = plsc.VectorSubcoreMesh(
        core_axis_name="c", subcore_axis_name="s",
        num_cores=NUM_CORES, num_subcores=NUM_SUBCORES)

    @functools.partial(
        pl.kernel,
        out_type=jax.ShapeDtypeStruct((CHUNK, HIST, EMBED_DIM), jnp.float32),
        mesh=mesh,
        compiler_params=pltpu.CompilerParams(use_tc_tiling_on_sc=False,
                                             skip_device_barrier=True),
        scratch_types=[
            pltpu.VMEM((RPW, HIST), jnp.int32),
            pltpu.VMEM((RPG, HIST, EMBED_DIM), jnp.float32),
            pltpu.VMEM((RPG, HIST, EMBED_DIM), jnp.float32),
            pltpu.SemaphoreType.DMA,
            pltpu.SemaphoreType.DMA,
            pltpu.SemaphoreType.DMA,
            pltpu.SemaphoreType.DMA,
        ],
    )
    def gather_kernel(idx_hbm, table_hbm, out_hbm,
                      idx_v, rows0, rows1, gsem0, gsem1, osem0, osem1):
        wid = lax.axis_index("s") * NUM_CORES + lax.axis_index("c")
        base = wid * RPW
        pltpu.sync_copy(idx_hbm.at[pl.ds(chunk * CHUNK + base, RPW)], idx_v)

        def start_gathers(g, rows, gsem):
            for r in range(RPG):
                pltpu.async_copy(
                    table_hbm.at[idx_v.at[g * RPG + r]],
                    rows.at[r],
                    gsem)

        def wait_gathers(rows, gsem):
            pltpu.make_async_copy(out_hbm.at[pl.ds(0, RPG)], rows, gsem).wait()

        def start_out(g, rows, osem):
            pltpu.async_copy(
                rows, out_hbm.at[pl.ds(base + g * RPG, RPG)], osem)

        def wait_out(rows, osem):
            pltpu.make_async_copy(rows, out_hbm.at[pl.ds(0, RPG)], osem).wait()

        start_gathers(0, rows0, gsem0)
        start_gathers(1, rows1, gsem1)
        wait_gathers(rows0, gsem0)
        start_out(0, rows0, osem0)

        def pair_body(p, carry):
            g = 2 * p + 1
            wait_out(rows0, osem0)
            start_gathers(g + 1, rows0, gsem0)
            wait_gathers(rows1, gsem1)
            start_out(g, rows1, osem1)

            wait_out(rows1, osem1)
            start_gathers(g + 2, rows1, gsem1)
            wait_gathers(rows0, gsem0)
            start_out(g + 1, rows0, osem0)
            return carry

        lax.fori_loop(0, (NG - 2) // 2, pair_body, 0)

        wait_out(rows0, osem0)
        wait_gathers(rows1, gsem1)
        start_out(NG - 1, rows1, osem1)
        wait_out(rows1, osem1)

    return gather_kernel


def kernel(indices, table, dummy):
    del dummy
    outs = [_make_kernel(c)(indices, table) for c in range(NCHUNK)]
    return jnp.concatenate(outs, axis=0)


# speedup vs baseline: 1.9530x; 1.0275x over previous
import functools

import jax
import jax.numpy as jnp
from jax import lax
from jax.experimental import pallas as pl
from jax.experimental.pallas import tpu as pltpu
from jax.experimental.pallas import tpu_sc as plsc

EMBED_DIM = 32
BATCH = 16384
HIST = 50

NUM_CORES = 2
NUM_SUBCORES = 16
NW = NUM_CORES * NUM_SUBCORES

NCHUNK = 4
CHUNK = BATCH // NCHUNK
RPW = CHUNK // NW
RPG = 16
NG = RPW // RPG


@functools.cache
def _make_kernel():
    mesh = plsc.VectorSubcoreMesh(
        core_axis_name="c", subcore_axis_name="s",
        num_cores=NUM_CORES, num_subcores=NUM_SUBCORES)

    @functools.partial(
        pl.kernel,
        out_type=jax.ShapeDtypeStruct((CHUNK, HIST, EMBED_DIM), jnp.float32),
        mesh=mesh,
        compiler_params=pltpu.CompilerParams(use_tc_tiling_on_sc=False,
                                             skip_device_barrier=True),
        scratch_types=[
            pltpu.VMEM((RPW, HIST), jnp.int32),
            pltpu.VMEM((RPG, HIST, EMBED_DIM), jnp.float32),
            pltpu.VMEM((RPG, HIST, EMBED_DIM), jnp.float32),
            pltpu.SemaphoreType.DMA,
            pltpu.SemaphoreType.DMA,
            pltpu.SemaphoreType.DMA,
            pltpu.SemaphoreType.DMA,
        ],
    )
    def gather_kernel(idx_hbm, table_hbm, out_hbm,
                      idx_v, rows0, rows1, gsem0, gsem1, osem0, osem1):
        wid = lax.axis_index("s") * NUM_CORES + lax.axis_index("c")
        base = wid * RPW
        pltpu.sync_copy(idx_hbm.at[pl.ds(base, RPW)], idx_v)

        def start_gathers(g, rows, gsem):
            for r in range(RPG):
                pltpu.async_copy(
                    table_hbm.at[idx_v.at[g * RPG + r]],
                    rows.at[r],
                    gsem)

        def wait_gathers(rows, gsem):
            pltpu.make_async_copy(out_hbm.at[pl.ds(0, RPG)], rows, gsem).wait()

        def start_out(g, rows, osem):
            pltpu.async_copy(
                rows, out_hbm.at[pl.ds(base + g * RPG, RPG)], osem)

        def wait_out(rows, osem):
            pltpu.make_async_copy(rows, out_hbm.at[pl.ds(0, RPG)], osem).wait()

        start_gathers(0, rows0, gsem0)
        start_gathers(1, rows1, gsem1)
        wait_gathers(rows0, gsem0)
        start_out(0, rows0, osem0)

        def pair_body(p, carry):
            g = 2 * p + 1
            wait_out(rows0, osem0)
            start_gathers(g + 1, rows0, gsem0)
            wait_gathers(rows1, gsem1)
            start_out(g, rows1, osem1)

            wait_out(rows1, osem1)
            start_gathers(g + 2, rows1, gsem1)
            wait_gathers(rows0, gsem0)
            start_out(g + 1, rows0, osem0)
            return carry

        lax.fori_loop(0, (NG - 2) // 2, pair_body, 0)

        wait_out(rows0, osem0)
        wait_gathers(rows1, gsem1)
        start_out(NG - 1, rows1, osem1)
        wait_out(rows1, osem1)

    return gather_kernel


def kernel(indices, table, dummy):
    del dummy
    k = _make_kernel()
    outs = [k(indices[c * CHUNK:(c + 1) * CHUNK], table)
            for c in range(NCHUNK)]
    return jnp.concatenate(outs, axis=0)
